# baseline (speedup 1.0000x reference)
import functools

import jax
import jax.numpy as jnp
from jax import lax
from jax.experimental import pallas as pl
from jax.experimental.pallas import tpu as pltpu
from jax.experimental.pallas import tpu_sc as plsc

N = 10000
F = 128
NC = 2
NS = 16
NW = NC * NS
CH = 128
NCHUNK = 80
EPW = NCHUNK * CH
EPAD = NW * EPW
NROWS = NW * NCHUNK
PADN = NROWS * CH - 320000
MROWS = 320000 // CH
DCH = 128
DUMP = N
NPAD = 10240
RPT = NPAD // NS
BR = 1000

_sc_mesh = plsc.VectorSubcoreMesh(
    core_axis_name="c", subcore_axis_name="s", num_cores=NC, num_subcores=NS)


@functools.partial(
    pl.kernel,
    out_type=jax.ShapeDtypeStruct((NPAD * F,), jnp.float32),
    mesh=_sc_mesh,
    scratch_types=[
        pltpu.VMEM((2 * EPW,), jnp.int32),
        pltpu.VMEM((NPAD,), jnp.float32),
        pltpu.VMEM((RPT,), jnp.float32),
        pltpu.VMEM((RPT,), jnp.float32),
        pltpu.VMEM((RPT,), jnp.float32),
        pltpu.VMEM((DCH * F,), jnp.float32),
        pltpu.VMEM_SHARED((NS, NPAD), jnp.float32),
    ],
    compiler_params=pltpu.CompilerParams(needs_layout_passes=False),
)
def _deg_sc(dst_hbm, tail_hbm, out_hbm, didx_v, hist_v, acc_v, tmp_v, dis_v,
            rowb_v, sh):
    cid = lax.axis_index("c")
    sid = lax.axis_index("s")
    one16 = jnp.ones((16,), jnp.float32)
    zero16 = jnp.zeros((16,), jnp.float32)

    @pl.when(cid == 0)
    def _():
        @pl.when(sid < NS - 1)
        def _():
            pltpu.sync_copy(dst_hbm.at[pl.ds(sid * 2 * EPW, 2 * EPW)],
                            didx_v)

        @pl.when(sid == NS - 1)
        def _():
            nmain = 2 * EPW - PADN
            pltpu.sync_copy(dst_hbm.at[pl.ds((NS - 1) * 2 * EPW, nmain)],
                            didx_v.at[pl.ds(0, nmain)])
            pltpu.sync_copy(tail_hbm, didx_v.at[pl.ds(nmain, PADN)])

        def _zh(k, carry):
            hist_v[pl.ds(k * 16, 16)] = zero16
            return carry
        lax.fori_loop(0, NPAD // 16, _zh, 0)

        def _hist(k, carry):
            idx16 = didx_v[pl.ds(k * 16, 16)]
            plsc.addupdate_scatter(hist_v, [idx16], one16)
            return carry
        lax.fori_loop(0, 2 * EPW // 16, _hist, 0)

        pltpu.sync_copy(hist_v, sh.at[sid])
        plsc.subcore_barrier()

        pltpu.sync_copy(sh.at[0, pl.ds(sid * RPT, RPT)], acc_v)
        for t in range(1, NS):
            pltpu.sync_copy(sh.at[t, pl.ds(sid * RPT, RPT)], tmp_v)

            def _add(k, carry):
                sl = pl.ds(k * 16, 16)
                acc_v[sl] = acc_v[sl] + tmp_v[sl]
                return carry
            lax.fori_loop(0, RPT // 16, _add, 0)

        def _rsq(k, carry):
            sl = pl.ds(k * 16, 16)
            d = acc_v[sl] + 1.0
            i = plsc.bitcast(d, jnp.int32)
            i = jnp.int32(0x5F3759DF) - lax.shift_right_logical(i, 1)
            y = plsc.bitcast(i, jnp.float32)
            y = y * (1.5 - 0.5 * d * y * y)
            y = y * (1.5 - 0.5 * d * y * y)
            y = y * (1.5 - 0.5 * d * y * y)
            dis_v[sl] = y
            return carry
        lax.fori_loop(0, RPT // 16, _rsq, 0)

        def _chunk(c, carry):
            def _row(r, carry2):
                v16 = plsc.load_gather(
                    dis_v, [jnp.full((16,), c * DCH + r, jnp.int32)])
                for j in range(F // 16):
                    rowb_v[pl.ds(r * F + j * 16, 16)] = v16
                return carry2
            lax.fori_loop(0, DCH, _row, 0)
            pltpu.sync_copy(
                rowb_v,
                out_hbm.at[pl.ds((sid * RPT + c * DCH) * F, DCH * F)])
            return carry
        lax.fori_loop(0, RPT // DCH, _chunk, 0)


@functools.partial(
    pl.kernel,
    out_type=jax.ShapeDtypeStruct((NC, NPAD, F), jnp.float32),
    mesh=_sc_mesh,
    scratch_types=[
        pltpu.VMEM((NCHUNK, CH), jnp.int32),
        pltpu.VMEM((NCHUNK, CH), jnp.int32),
        pltpu.VMEM((CH, F), jnp.float32),
        pltpu.VMEM_SHARED((NPAD, F), jnp.float32),
        pltpu.SemaphoreType.DMA,
    ],
)
def _edge_sc(src_hbm, dst_hbm, stail_hbm, dtail_hbm, hs_hbm, out_hbm,
             sidx_v, didx_v, rows_v, acc_sh, sem):
    cid = lax.axis_index("c")
    sid = lax.axis_index("s")
    zero16 = jnp.zeros((16,), jnp.float32)

    def _zrow(i, carry):
        for j in range(F // 16):
            rows_v[i, pl.ds(j * 16, 16)] = zero16
        return carry
    lax.fori_loop(0, CH, _zrow, 0)

    def _zacc(t, carry):
        pltpu.sync_copy(rows_v, acc_sh.at[pl.ds(sid * RPT + t * CH, CH)])
        return carry
    lax.fori_loop(0, RPT // CH, _zacc, 0)

    wid = cid * NS + sid

    @pl.when(wid < NW - 1)
    def _():
        pltpu.sync_copy(src_hbm.at[pl.ds(wid * NCHUNK, NCHUNK)], sidx_v)
        pltpu.sync_copy(dst_hbm.at[pl.ds(wid * NCHUNK, NCHUNK)], didx_v)

    @pl.when(wid == NW - 1)
    def _():
        pltpu.sync_copy(stail_hbm, sidx_v)
        pltpu.sync_copy(dtail_hbm, didx_v)
    plsc.subcore_barrier()

    def _go(g, carry):
        pltpu.async_copy(hs_hbm.at[sidx_v.at[g]], rows_v, sem).wait()
        pltpu.sync_copy(rows_v, acc_sh.at[didx_v.at[g]], add=True)
        return carry
    lax.fori_loop(0, NCHUNK, _go, 0)
    plsc.subcore_barrier()
    pltpu.sync_copy(acc_sh.at[pl.ds(sid * RPT, RPT)],
                    out_hbm.at[cid, pl.ds(sid * RPT, RPT)])


def _prep_body(x_ref, w_ref, b_ref, dis_ref, hs_ref, base_ref):
    dis = dis_ref[...]
    h = jnp.dot(x_ref[...], w_ref[...],
                preferred_element_type=jnp.float32)
    hs = h * dis
    hs_ref[...] = hs
    base_ref[...] = hs * dis + b_ref[...]


_prep = pl.pallas_call(
    _prep_body,
    grid=(N // BR,),
    in_specs=[
        pl.BlockSpec((BR, F), lambda i: (i, 0)),
        pl.BlockSpec((F, F), lambda i: (0, 0)),
        pl.BlockSpec((1, F), lambda i: (0, 0)),
        pl.BlockSpec((BR, F), lambda i: (i, 0)),
    ],
    out_specs=[pl.BlockSpec((BR, F), lambda i: (i, 0)),
               pl.BlockSpec((BR, F), lambda i: (i, 0))],
    out_shape=[jax.ShapeDtypeStruct((N, F), jnp.float32),
               jax.ShapeDtypeStruct((N, F), jnp.float32)],
)


def _comb_body(part_ref, dis_ref, base_ref, out_ref):
    out_ref[...] = ((part_ref[0] + part_ref[1]) * dis_ref[...]
                    + base_ref[...])


_comb = pl.pallas_call(
    _comb_body,
    grid=(N // BR,),
    in_specs=[
        pl.BlockSpec((NC, BR, F), lambda i: (0, i, 0)),
        pl.BlockSpec((BR, F), lambda i: (i, 0)),
        pl.BlockSpec((BR, F), lambda i: (i, 0)),
    ],
    out_specs=pl.BlockSpec((BR, F), lambda i: (i, 0)),
    out_shape=jax.ShapeDtypeStruct((N, F), jnp.float32),
)


def kernel(x, edge_index, W, b):
    src = edge_index[0].astype(jnp.int32)
    dst = edge_index[1].astype(jnp.int32)
    spad = jnp.arange(PADN, dtype=jnp.int32) % N
    dpad = DUMP + jnp.arange(PADN, dtype=jnp.int32) % (NPAD - DUMP)
    src2 = src.reshape(MROWS, CH)
    dst2 = dst.reshape(MROWS, CH)
    ntail = NROWS * CH - (NW - 1) * NCHUNK * CH - 320000 + PADN
    nmain_tail = NCHUNK * CH - PADN
    stail = jnp.concatenate(
        [src[-nmain_tail:], spad]).reshape(NCHUNK, CH)
    dtail = jnp.concatenate(
        [dst[-nmain_tail:], dpad]).reshape(NCHUNK, CH)
    dis = _deg_sc(dst, dpad).reshape(NPAD, F)
    hs, base = _prep(x, W, b.reshape(1, F), dis)
    part = _edge_sc(src2, dst2, stail, dtail, hs)
    return _comb(part, dis, base)


# speedup vs baseline: 1.0051x; 1.0051x over previous
import functools

import jax
import jax.numpy as jnp
from jax import lax
from jax.experimental import pallas as pl
from jax.experimental.pallas import tpu as pltpu
from jax.experimental.pallas import tpu_sc as plsc

N = 10000
F = 128
NC = 2
NS = 16
NW = NC * NS
CH = 128
NCHUNK = 80
EPW = NCHUNK * CH
EPAD = NW * EPW
NROWS = NW * NCHUNK
DCH = 128
DUMP = N
NPAD = 10240
RPT = NPAD // NS
BR = 1000

_sc_mesh = plsc.VectorSubcoreMesh(
    core_axis_name="c", subcore_axis_name="s", num_cores=NC, num_subcores=NS)


@functools.partial(
    pl.kernel,
    out_type=jax.ShapeDtypeStruct((NPAD * F,), jnp.float32),
    mesh=_sc_mesh,
    scratch_types=[
        pltpu.VMEM((2 * EPW,), jnp.int32),
        pltpu.VMEM((NPAD,), jnp.float32),
        pltpu.VMEM((RPT,), jnp.float32),
        pltpu.VMEM((RPT,), jnp.float32),
        pltpu.VMEM((RPT,), jnp.float32),
        pltpu.VMEM((DCH * F,), jnp.float32),
        pltpu.VMEM_SHARED((NS, NPAD), jnp.float32),
    ],
    compiler_params=pltpu.CompilerParams(needs_layout_passes=False),
)
def _deg_sc(dst_hbm, out_hbm, didx_v, hist_v, acc_v, tmp_v, dis_v, rowb_v, sh):
    cid = lax.axis_index("c")
    sid = lax.axis_index("s")
    one16 = jnp.ones((16,), jnp.float32)
    zero16 = jnp.zeros((16,), jnp.float32)

    @pl.when(cid == 0)
    def _():
        pltpu.sync_copy(dst_hbm.at[pl.ds(sid * 2 * EPW, 2 * EPW)], didx_v)

        def _zh(k, carry):
            hist_v[pl.ds(k * 16, 16)] = zero16
            return carry
        lax.fori_loop(0, NPAD // 16, _zh, 0)

        def _hist(k, carry):
            idx16 = didx_v[pl.ds(k * 16, 16)]
            plsc.addupdate_scatter(hist_v, [idx16], one16)
            return carry
        lax.fori_loop(0, 2 * EPW // 16, _hist, 0)

        pltpu.sync_copy(hist_v, sh.at[sid])
        plsc.subcore_barrier()

        pltpu.sync_copy(sh.at[0, pl.ds(sid * RPT, RPT)], acc_v)
        for t in range(1, NS):
            pltpu.sync_copy(sh.at[t, pl.ds(sid * RPT, RPT)], tmp_v)

            def _add(k, carry):
                sl = pl.ds(k * 16, 16)
                acc_v[sl] = acc_v[sl] + tmp_v[sl]
                return carry
            lax.fori_loop(0, RPT // 16, _add, 0)

        def _rsq(k, carry):
            sl = pl.ds(k * 16, 16)
            d = acc_v[sl] + 1.0
            i = plsc.bitcast(d, jnp.int32)
            i = jnp.int32(0x5F3759DF) - lax.shift_right_logical(i, 1)
            y = plsc.bitcast(i, jnp.float32)
            y = y * (1.5 - 0.5 * d * y * y)
            y = y * (1.5 - 0.5 * d * y * y)
            y = y * (1.5 - 0.5 * d * y * y)
            dis_v[sl] = y
            return carry
        lax.fori_loop(0, RPT // 16, _rsq, 0)

        def _chunk(c, carry):
            def _row(r, carry2):
                v16 = plsc.load_gather(
                    dis_v, [jnp.full((16,), c * DCH + r, jnp.int32)])
                for j in range(F // 16):
                    rowb_v[pl.ds(r * F + j * 16, 16)] = v16
                return carry2
            lax.fori_loop(0, DCH, _row, 0)
            pltpu.sync_copy(
                rowb_v,
                out_hbm.at[pl.ds((sid * RPT + c * DCH) * F, DCH * F)])
            return carry
        lax.fori_loop(0, RPT // DCH, _chunk, 0)


@functools.partial(
    pl.kernel,
    out_type=jax.ShapeDtypeStruct((NC, NPAD, F), jnp.float32),
    mesh=_sc_mesh,
    scratch_types=[
        pltpu.VMEM((NCHUNK, CH), jnp.int32),
        pltpu.VMEM((NCHUNK, CH), jnp.int32),
        pltpu.VMEM((CH, F), jnp.float32),
        pltpu.VMEM_SHARED((NPAD, F), jnp.float32),
        pltpu.SemaphoreType.DMA,
    ],
)
def _edge_sc(src_hbm, dst_hbm, hs_hbm, out_hbm,
             sidx_v, didx_v, rows_v, acc_sh, sem):
    cid = lax.axis_index("c")
    sid = lax.axis_index("s")
    zero16 = jnp.zeros((16,), jnp.float32)

    def _zrow(i, carry):
        for j in range(F // 16):
            rows_v[i, pl.ds(j * 16, 16)] = zero16
        return carry
    lax.fori_loop(0, CH, _zrow, 0)

    def _zacc(t, carry):
        pltpu.sync_copy(rows_v, acc_sh.at[pl.ds(sid * RPT + t * CH, CH)])
        return carry
    lax.fori_loop(0, RPT // CH, _zacc, 0)

    choff = (cid * NS + sid) * NCHUNK
    pltpu.sync_copy(src_hbm.at[pl.ds(choff, NCHUNK)], sidx_v)
    pltpu.sync_copy(dst_hbm.at[pl.ds(choff, NCHUNK)], didx_v)
    plsc.subcore_barrier()

    def _go(g, carry):
        pltpu.async_copy(hs_hbm.at[sidx_v.at[g]], rows_v, sem).wait()
        pltpu.sync_copy(rows_v, acc_sh.at[didx_v.at[g]], add=True)
        return carry
    lax.fori_loop(0, NCHUNK, _go, 0)
    plsc.subcore_barrier()
    pltpu.sync_copy(acc_sh.at[pl.ds(sid * RPT, RPT)],
                    out_hbm.at[cid, pl.ds(sid * RPT, RPT)])


def _prep_body(x_ref, w_ref, b_ref, dis_ref, hs_ref, base_ref):
    dis = dis_ref[...]
    h = jnp.dot(x_ref[...], w_ref[...],
                preferred_element_type=jnp.float32)
    hs = h * dis
    hs_ref[...] = hs
    base_ref[...] = hs * dis + b_ref[...]


_prep = pl.pallas_call(
    _prep_body,
    grid=(N // BR,),
    in_specs=[
        pl.BlockSpec((BR, F), lambda i: (i, 0)),
        pl.BlockSpec((F, F), lambda i: (0, 0)),
        pl.BlockSpec((1, F), lambda i: (0, 0)),
        pl.BlockSpec((BR, F), lambda i: (i, 0)),
    ],
    out_specs=[pl.BlockSpec((BR, F), lambda i: (i, 0)),
               pl.BlockSpec((BR, F), lambda i: (i, 0))],
    out_shape=[jax.ShapeDtypeStruct((N, F), jnp.float32),
               jax.ShapeDtypeStruct((N, F), jnp.float32)],
)


def _comb_body(part_ref, dis_ref, base_ref, out_ref):
    out_ref[...] = ((part_ref[0] + part_ref[1]) * dis_ref[...]
                    + base_ref[...])


_comb = pl.pallas_call(
    _comb_body,
    grid=(N // BR,),
    in_specs=[
        pl.BlockSpec((NC, BR, F), lambda i: (0, i, 0)),
        pl.BlockSpec((BR, F), lambda i: (i, 0)),
        pl.BlockSpec((BR, F), lambda i: (i, 0)),
    ],
    out_specs=pl.BlockSpec((BR, F), lambda i: (i, 0)),
    out_shape=jax.ShapeDtypeStruct((N, F), jnp.float32),
)


def kernel(x, edge_index, W, b):
    src = edge_index[0].astype(jnp.int32)
    dst = edge_index[1].astype(jnp.int32)
    pad = NROWS * CH - src.shape[0]
    src2 = jnp.concatenate(
        [src, jnp.arange(pad, dtype=jnp.int32) % N]).reshape(NROWS, CH)
    dump = DUMP + jnp.arange(pad, dtype=jnp.int32) % (NPAD - DUMP)
    dst_pad = jnp.concatenate([dst, dump])
    dst2 = dst_pad.reshape(NROWS, CH)
    dis = _deg_sc(dst_pad).reshape(NPAD, F)
    hs, base = _prep(x, W, b.reshape(1, F), dis)
    part = _edge_sc(src2, dst2, hs)
    return _comb(part, dis, base)
